# Initial kernel scaffold; baseline (speedup 1.0000x reference)
import jax
import jax.numpy as jnp
from jax.experimental import pallas as pl


def kernel(user_id, sku_id, user_features, sku_features, emb_user, emb_sku, bias_user, bias_sku, W_user, b_user, W_sku, b_sku):
    raise NotImplementedError("write your pallas kernel here")


# speedup vs baseline: 4.4542x; 4.4542x over previous
import functools

import jax
import jax.numpy as jnp
from jax import lax
from jax.experimental import pallas as pl
from jax.experimental.pallas import tpu as pltpu
from jax.experimental.pallas import tpu_sc as plsc

B = 16384
D = 64
DA = 80


def _sc_gather_body(nc, bpw,
                    tab_u_hbm, tab_s_hbm, uid_hbm, sid_hbm,
                    xe_u_hbm, xe_s_hbm,
                    idx_u, idx_s, rows_u, rows_s, sem_u, sem_s):
    wid = lax.axis_index("s") * nc + lax.axis_index("c")
    base = wid * bpw
    pltpu.sync_copy(uid_hbm.at[pl.ds(base, bpw)], idx_u)
    pltpu.sync_copy(sid_hbm.at[pl.ds(base, bpw)], idx_s)
    cp_u = pltpu.async_copy(tab_u_hbm.at[idx_u], rows_u, sem_u)
    cp_s = pltpu.async_copy(tab_s_hbm.at[idx_s], rows_s, sem_s)
    cp_u.wait()
    pltpu.sync_copy(rows_u, xe_u_hbm.at[pl.ds(base, bpw)])
    cp_s.wait()
    pltpu.sync_copy(rows_s, xe_s_hbm.at[pl.ds(base, bpw)])


@functools.cache
def _make_sc_gather():
    info = plsc.get_sparse_core_info()
    nc, ns = info.num_cores, info.num_subcores
    nw = nc * ns
    bpw = B // nw
    mesh = plsc.VectorSubcoreMesh(core_axis_name="c", subcore_axis_name="s",
                                  num_cores=nc)
    return pl.kernel(
        functools.partial(_sc_gather_body, nc, bpw),
        out_type=(
            jax.ShapeDtypeStruct((B, DA), jnp.float32),
            jax.ShapeDtypeStruct((B, DA), jnp.float32),
        ),
        mesh=mesh,
        scratch_types=[
            pltpu.VMEM((bpw,), jnp.int32),
            pltpu.VMEM((bpw,), jnp.int32),
            pltpu.VMEM((bpw, DA), jnp.float32),
            pltpu.VMEM((bpw, DA), jnp.float32),
            pltpu.SemaphoreType.DMA,
            pltpu.SemaphoreType.DMA,
        ],
        compiler_params=pltpu.CompilerParams(use_tc_tiling_on_sc=False),
        name="sc_embed_gather",
    )


BLK = 2048


def _tc_body(uf_ref, sf_ref, wu_ref, bu_ref, ws_ref, bs_ref,
             xeu_ref, xes_ref, out_ref):
    xfu = jnp.maximum(
        jnp.dot(uf_ref[...], wu_ref[...],
                preferred_element_type=jnp.float32) + bu_ref[...], 0.0)
    xfs = jnp.maximum(
        jnp.dot(sf_ref[...], ws_ref[...],
                preferred_element_type=jnp.float32) + bs_ref[...], 0.0)
    eu = xeu_ref[:, :D] + xfu
    es = xes_ref[:, :D] + xfs
    comb = jnp.sum(eu * es, axis=1)
    xb = xeu_ref[:, D] + xes_ref[:, D]
    out_ref[...] = 4.0 * jax.nn.sigmoid(xb + comb) + 1.0


def _tc_dense(uf, sf, wu, bu, ws, bs, xeu, xes):
    nblk = B // BLK
    return pl.pallas_call(
        _tc_body,
        grid=(nblk,),
        in_specs=[
            pl.BlockSpec((BLK, uf.shape[1]), lambda i: (i, 0)),
            pl.BlockSpec((BLK, sf.shape[1]), lambda i: (i, 0)),
            pl.BlockSpec(wu.shape, lambda i: (0, 0)),
            pl.BlockSpec(bu.shape, lambda i: (0, 0)),
            pl.BlockSpec(ws.shape, lambda i: (0, 0)),
            pl.BlockSpec(bs.shape, lambda i: (0, 0)),
            pl.BlockSpec((BLK, DA), lambda i: (i, 0)),
            pl.BlockSpec((BLK, DA), lambda i: (i, 0)),
        ],
        out_specs=pl.BlockSpec((BLK,), lambda i: (i,)),
        out_shape=jax.ShapeDtypeStruct((B,), jnp.float32),
        compiler_params=pltpu.CompilerParams(
            dimension_semantics=("arbitrary",),
        ),
        name="tc_rating_dense",
    )(uf, sf, wu, bu, ws, bs, xeu, xes)


def _augment(emb, bias):
    v = emb.shape[0]
    pad = jnp.zeros((v, DA - D - 1), dtype=emb.dtype)
    return jnp.concatenate([emb, bias, pad], axis=1)


def kernel(user_id, sku_id, user_features, sku_features, emb_user, emb_sku,
           bias_user, bias_sku, W_user, b_user, W_sku, b_sku):
    uid = user_id[:, 0].astype(jnp.int32)
    sid = sku_id[:, 0].astype(jnp.int32)
    tab_u = _augment(emb_user, bias_user)
    tab_s = _augment(emb_sku, bias_sku)
    xe_u, xe_s = _make_sc_gather()(tab_u, tab_s, uid, sid)
    return _tc_dense(user_features, sku_features,
                     W_user, b_user.reshape(1, D),
                     W_sku, b_sku.reshape(1, D),
                     xe_u, xe_s)
